# initial kernel scaffold (unmeasured)
import jax
import jax.numpy as jnp
from jax import lax
from jax.experimental import pallas as pl
from jax.experimental.pallas import tpu as pltpu


def kernel(
    x,
):
    def body(*refs):
        pass

    out_shape = jax.ShapeDtypeStruct(..., jnp.float32)
    return pl.pallas_call(body, out_shape=out_shape)(...)


# baseline (device time: 19225 ns/iter reference)
import jax
import jax.numpy as jnp
from jax import lax
from jax.experimental import pallas as pl
from jax.experimental.pallas import tpu as pltpu

N_Z = 4


def kernel(x):
    m_per, n = x.shape

    def body(x_ref, out_ref, comm_ref, send_sems, recv_sems):
        my_x = lax.axis_index("x")
        my_y = lax.axis_index("y")
        my_z = lax.axis_index("z")
        left = (my_z - 1) % N_Z
        right = (my_z + 1) % N_Z

        barrier_sem = pltpu.get_barrier_semaphore()
        for nbr in [left, right]:
            pl.semaphore_signal(
                barrier_sem, inc=1,
                device_id=(my_x, my_y, nbr),
                device_id_type=pl.DeviceIdType.MESH,
            )
        pl.semaphore_wait(barrier_sem, 2)

        out_ref[pl.ds(my_z * m_per, m_per), :] = x_ref[:, :]
        comm_ref[0, :, :] = x_ref[:, :]

        for h in range(N_Z - 1):
            send_slot = h % 2
            recv_slot = (h + 1) % 2
            rdma = pltpu.make_async_remote_copy(
                src_ref=comm_ref.at[send_slot],
                dst_ref=comm_ref.at[recv_slot],
                send_sem=send_sems.at[send_slot],
                recv_sem=recv_sems.at[recv_slot],
                device_id=(my_x, my_y, right),
                device_id_type=pl.DeviceIdType.MESH,
            )
            rdma.start()
            rdma.wait()

            origin = (my_z - h - 1) % N_Z
            out_ref[pl.ds(origin * m_per, m_per), :] = comm_ref[recv_slot, :, :]

    return pl.pallas_call(
        body,
        out_shape=jax.ShapeDtypeStruct((N_Z * m_per, n), x.dtype),
        in_specs=[pl.BlockSpec(memory_space=pltpu.VMEM)],
        out_specs=pl.BlockSpec(memory_space=pltpu.VMEM),
        scratch_shapes=[
            pltpu.VMEM((2, m_per, n), x.dtype),
            pltpu.SemaphoreType.DMA((2,)),
            pltpu.SemaphoreType.DMA((2,)),
        ],
        compiler_params=pltpu.CompilerParams(collective_id=0),
    )(x)


# device time: 15831 ns/iter; 1.2144x vs baseline; 1.2144x over previous
import jax
import jax.numpy as jnp
from jax import lax
from jax.experimental import pallas as pl
from jax.experimental.pallas import tpu as pltpu

N_Z = 4


def kernel(x):
    m_per, n = x.shape
    half = m_per // 2

    def body(x_ref, out_ref, send_sems, recv_sems):
        my_x = lax.axis_index("x")
        my_y = lax.axis_index("y")
        my_z = lax.axis_index("z")
        left = (my_z - 1) % N_Z
        right = (my_z + 1) % N_Z

        barrier_sem = pltpu.get_barrier_semaphore()
        for nbr in [left, right]:
            pl.semaphore_signal(
                barrier_sem, inc=1,
                device_id=(my_x, my_y, nbr),
                device_id_type=pl.DeviceIdType.MESH,
            )
        pl.semaphore_wait(barrier_sem, 2)

        s1p = pltpu.make_async_remote_copy(
            src_ref=x_ref,
            dst_ref=out_ref.at[pl.ds(my_z * m_per, m_per), :],
            send_sem=send_sems.at[0],
            recv_sem=recv_sems.at[0],
            device_id=(my_x, my_y, right),
            device_id_type=pl.DeviceIdType.MESH,
        )
        s1m = pltpu.make_async_remote_copy(
            src_ref=x_ref,
            dst_ref=out_ref.at[pl.ds(my_z * m_per, m_per), :],
            send_sem=send_sems.at[1],
            recv_sem=recv_sems.at[1],
            device_id=(my_x, my_y, left),
            device_id_type=pl.DeviceIdType.MESH,
        )
        s1p.start()
        s1m.start()

        out_ref[pl.ds(my_z * m_per, m_per), :] = x_ref[:, :]

        s1p.wait_recv()
        s2p = pltpu.make_async_remote_copy(
            src_ref=out_ref.at[pl.ds(left * m_per, half), :],
            dst_ref=out_ref.at[pl.ds(left * m_per, half), :],
            send_sem=send_sems.at[2],
            recv_sem=recv_sems.at[2],
            device_id=(my_x, my_y, right),
            device_id_type=pl.DeviceIdType.MESH,
        )
        s2p.start()

        s1m.wait_recv()
        s2m = pltpu.make_async_remote_copy(
            src_ref=out_ref.at[pl.ds(right * m_per + half, half), :],
            dst_ref=out_ref.at[pl.ds(right * m_per + half, half), :],
            send_sem=send_sems.at[3],
            recv_sem=recv_sems.at[3],
            device_id=(my_x, my_y, left),
            device_id_type=pl.DeviceIdType.MESH,
        )
        s2m.start()

        s2p.wait_recv()
        s2m.wait_recv()

        s1p.wait_send()
        s1m.wait_send()
        s2p.wait_send()
        s2m.wait_send()

    return pl.pallas_call(
        body,
        out_shape=jax.ShapeDtypeStruct((N_Z * m_per, n), x.dtype),
        in_specs=[pl.BlockSpec(memory_space=pltpu.VMEM)],
        out_specs=pl.BlockSpec(memory_space=pltpu.VMEM),
        scratch_shapes=[
            pltpu.SemaphoreType.DMA((4,)),
            pltpu.SemaphoreType.DMA((4,)),
        ],
        compiler_params=pltpu.CompilerParams(collective_id=0),
    )(x)
